# speedup vs baseline: 4.2565x; 4.2565x over previous
import functools

import jax
import jax.numpy as jnp
from jax import lax
from jax.experimental import pallas as pl
from jax.experimental.pallas import tpu as pltpu
from jax.experimental.pallas import tpu_sc as plsc

EMBED_RATIO = 11.313708498984761
VOCAB = 100000
D_MODEL = 128
SEQ_LEN = 200
BATCH = 4096

NUM_CORES = 2
NUM_SUBCORES = 16
NUM_WORKERS = NUM_CORES * NUM_SUBCORES
SEQ_PER_WORKER = BATCH // NUM_WORKERS
GATHER_CHUNK = 100
CHUNKS_PER_SEQ = SEQ_LEN // GATHER_CHUNK
LANES = 16


def _embed_body(inputs_hbm, pe_hbm, table_hbm, out_hbm,
                pe_v, idx_v, rows_v, sem):
    wid = lax.axis_index("s") * NUM_CORES + lax.axis_index("c")

    pltpu.sync_copy(pe_hbm, pe_v)

    def seq_body(t, _):
        b = wid * SEQ_PER_WORKER + t
        pltpu.sync_copy(inputs_hbm.at[b], idx_v)
        cps = []
        for c in range(CHUNKS_PER_SEQ):
            cps.append(pltpu.async_copy(
                table_hbm.at[idx_v.at[c]],
                rows_v.at[pl.ds(c * GATHER_CHUNK, GATHER_CHUNK)],
                sem))
        for cp in cps:
            cp.wait()

        def row_body(r, _):
            for k in range(D_MODEL // LANES):
                sl = pl.ds(k * LANES, LANES)
                rows_v[r, sl] = rows_v[r, sl] * EMBED_RATIO + pe_v[r, sl]
            return ()

        lax.fori_loop(0, SEQ_LEN, row_body, ())

        pltpu.sync_copy(rows_v, out_hbm.at[b])
        return ()

    lax.fori_loop(0, SEQ_PER_WORKER, seq_body, ())


@jax.jit
def kernel(inputs, table, pe):
    inputs3 = inputs.reshape(BATCH, CHUNKS_PER_SEQ, GATHER_CHUNK)
    pe2 = pe.reshape(SEQ_LEN, D_MODEL)

    mesh = plsc.VectorSubcoreMesh(
        core_axis_name="c", subcore_axis_name="s",
        num_cores=NUM_CORES, num_subcores=NUM_SUBCORES)

    out = pl.kernel(
        _embed_body,
        out_type=jax.ShapeDtypeStruct((BATCH, SEQ_LEN, D_MODEL),
                                      jnp.float32),
        mesh=mesh,
        scratch_types=[
            pltpu.VMEM((SEQ_LEN, D_MODEL), jnp.float32),
            pltpu.VMEM((CHUNKS_PER_SEQ, GATHER_CHUNK), jnp.int32),
            pltpu.VMEM((SEQ_LEN, D_MODEL), jnp.float32),
            pltpu.SemaphoreType.DMA,
        ],
    )(inputs3, pe2, table)
    return out


# speedup vs baseline: 8.9693x; 2.1072x over previous
import jax
import jax.numpy as jnp
from jax import lax
from jax.experimental import pallas as pl
from jax.experimental.pallas import tpu as pltpu
from jax.experimental.pallas import tpu_sc as plsc

EMBED_RATIO = 11.313708498984761
D_MODEL = 128
SEQ_LEN = 200
BATCH = 4096

NUM_CORES = 2
NUM_SUBCORES = 16
NUM_WORKERS = NUM_CORES * NUM_SUBCORES
SEQ_PER_WORKER = BATCH // NUM_WORKERS
CHUNK = 40
CHUNKS_PER_SEQ = SEQ_LEN // CHUNK
NQ = SEQ_PER_WORKER * CHUNKS_PER_SEQ
NBUF = 2 * CHUNKS_PER_SEQ
LOOKAHEAD = CHUNKS_PER_SEQ
LANES = 16


def _embed_body(inputs_hbm, pe_hbm, table_hbm, out_hbm,
                pe_v, idx_v, rows_v, gsems, wsems):
    wid = lax.axis_index("s") * NUM_CORES + lax.axis_index("c")
    b0 = wid * SEQ_PER_WORKER

    pltpu.sync_copy(pe_hbm, pe_v)
    pltpu.sync_copy(
        inputs_hbm.at[pl.ds(b0 * SEQ_LEN, SEQ_PER_WORKER * SEQ_LEN)], idx_v)

    def issue_gather(q, ph):
        return pltpu.async_copy(
            table_hbm.at[idx_v.at[pl.ds(q * CHUNK, CHUNK)]],
            rows_v.at[ph], gsems[ph])

    for q in range(LOOKAHEAD):
        issue_gather(q, q % NBUF)

    def outer(g):
        for ph in range(NBUF):
            q = g + ph
            h = ph % CHUNKS_PER_SEQ
            t = g // CHUNKS_PER_SEQ + ph // CHUNKS_PER_SEQ

            phn = (ph + LOOKAHEAD) % NBUF
            hn = phn % CHUNKS_PER_SEQ

            @pl.when(q + LOOKAHEAD < NQ)
            def _():
                @pl.when(q >= NBUF - LOOKAHEAD)
                def _():
                    pltpu.make_async_copy(
                        rows_v.at[phn],
                        out_hbm.at[b0, pl.ds(0, CHUNK)],
                        wsems[phn]).wait()
                issue_gather(q + LOOKAHEAD, phn)

            pltpu.make_async_copy(
                table_hbm.at[idx_v.at[pl.ds(q * CHUNK, CHUNK)]],
                rows_v.at[ph], gsems[ph]).wait()

            buf = rows_v.at[ph]

            def row_body(r, _):
                for k in range(D_MODEL // LANES):
                    sl = pl.ds(k * LANES, LANES)
                    buf[r, sl] = buf[r, sl] * EMBED_RATIO \
                        + pe_v[h * CHUNK + r, sl]
                return ()

            lax.fori_loop(0, CHUNK, row_body, ())

            pltpu.async_copy(
                buf, out_hbm.at[b0 + t, pl.ds(h * CHUNK, CHUNK)], wsems[ph])

    def outer_body(i, carry):
        outer(i * NBUF)
        return carry

    lax.fori_loop(0, NQ // NBUF, outer_body, ())

    for ph in range(NBUF):
        pltpu.make_async_copy(
            rows_v.at[ph], out_hbm.at[b0, pl.ds(0, CHUNK)],
            wsems[ph]).wait()


@jax.jit
def kernel(inputs, table, pe):
    inputs_flat = inputs.reshape(BATCH * SEQ_LEN)
    pe2 = pe.reshape(SEQ_LEN, D_MODEL)

    mesh = plsc.VectorSubcoreMesh(
        core_axis_name="c", subcore_axis_name="s",
        num_cores=NUM_CORES, num_subcores=NUM_SUBCORES)

    out = pl.kernel(
        _embed_body,
        out_type=jax.ShapeDtypeStruct((BATCH, SEQ_LEN, D_MODEL),
                                      jnp.float32),
        mesh=mesh,
        scratch_types=[
            pltpu.VMEM((SEQ_LEN, D_MODEL), jnp.float32),
            pltpu.VMEM((SEQ_PER_WORKER * SEQ_LEN,), jnp.int32),
            pltpu.VMEM((NBUF, CHUNK, D_MODEL), jnp.float32),
            [pltpu.SemaphoreType.DMA] * NBUF,
            [pltpu.SemaphoreType.DMA] * NBUF,
        ],
    )(inputs_flat, pe2, table)
    return out
